# baseline (speedup 1.0000x reference)
import functools

import jax
import jax.numpy as jnp
import numpy as np
from jax import lax
from jax.experimental import pallas as pl
from jax.experimental.pallas import tpu as pltpu
from jax.experimental.pallas import tpu_sc as plsc

_B, _N, _D, _M, _K = 2, 2048, 128, 16, 16
_RK = 256
_RE = 256
_NC, _NS = 2, 16
_NW = _NC * _NS
_TOT = _B * _N * _K
_PW = _TOT // _NW
_CH = 128


def _make_knn_body(boff):
    def _knn_body(ci_ref, ct_ref, idx_ref, dk_ref):
        ci = ci_ref[0]
        ct = ct_ref[0]
        d = None
        for a in range(3):
            diff = ci[:, a:a + 1] - ct[a:a + 1, :]
            sq = diff * diff
            d = sq if d is None else d + sq
        iota = lax.broadcasted_iota(jnp.int32, (_RK, _N), 1)
        bits = lax.bitcast_convert_type(d, jnp.int32)
        p = lax.bitcast_convert_type(
            ((bits & jnp.int32(~0x7FF)) | iota) + jnp.int32(0x00800000),
            jnp.float32)
        sentinel = jnp.float32(np.inf)
        for k in range(_K):
            w = jnp.min(p, axis=1, keepdims=True)
            wb = lax.bitcast_convert_type(w, jnp.int32) \
                - jnp.int32(0x00800000)
            idx_ref[0, :, k:k + 1] = (wb & jnp.int32(0x7FF)) + boff
            dk_ref[0, :, k:k + 1] = lax.bitcast_convert_type(
                wb & jnp.int32(~0x7FF), jnp.float32)
            p = jnp.where(p == w, sentinel, p)

    return _knn_body


def _knn_call_args():
    grid = (1, _N // _RK)
    return dict(
        grid=grid,
        in_specs=[
            pl.BlockSpec((1, _RK, 3), lambda b, i: (b, i, 0)),
            pl.BlockSpec((1, 3, _N), lambda b, i: (b, 0, 0)),
        ],
        out_specs=[
            pl.BlockSpec((1, _RK, _K), lambda b, i: (b, i, 0)),
            pl.BlockSpec((1, _RK, _K), lambda b, i: (b, i, 0)),
        ],
        out_shape=[
            jax.ShapeDtypeStruct((1, _N, _K), jnp.int32),
            jax.ShapeDtypeStruct((1, _N, _K), jnp.float32),
        ],
    )


def _sc_gather(idx_flat, table):
    tot = idx_flat.shape[0]
    pw = tot // _NW
    mesh = plsc.VectorSubcoreMesh(core_axis_name="c", subcore_axis_name="s")

    @functools.partial(
        pl.kernel,
        mesh=mesh,
        out_type=jax.ShapeDtypeStruct((tot, _D), jnp.float32),
        scratch_types=[
            pltpu.VMEM((pw,), jnp.int32),
            pltpu.VMEM((_CH, _D), jnp.float32),
            pltpu.VMEM((_CH, _D), jnp.float32),
            pltpu.SemaphoreType.DMA,
            pltpu.SemaphoreType.DMA,
            pltpu.SemaphoreType.DMA,
            pltpu.SemaphoreType.DMA,
        ],
    )
    def gk(idx_hbm, tab_hbm, out_hbm, idx_v, rows0, rows1, g0, g1, w0, w1):
        wid = lax.axis_index("s") * _NC + lax.axis_index("c")
        base = wid * pw
        pltpu.sync_copy(idx_hbm.at[pl.ds(base, pw)], idx_v)
        bufs, gsems, wsems = (rows0, rows1), (g0, g1), (w0, w1)
        nch = pw // _CH
        gathers = [None] * nch
        wbacks = [None] * nch

        def start_gather(j):
            s = j & 1
            if j >= 2:
                wbacks[j - 2].wait()
            gathers[j] = pltpu.async_copy(
                tab_hbm.at[idx_v.at[pl.ds(j * _CH, _CH)]], bufs[s], gsems[s])

        start_gather(0)
        for j in range(nch):
            if j + 1 < nch:
                start_gather(j + 1)
            gathers[j].wait()
            wbacks[j] = pltpu.async_copy(
                bufs[j & 1], out_hbm.at[pl.ds(base + j * _CH, _CH)],
                wsems[j & 1])
        wbacks[nch - 2].wait()
        wbacks[nch - 1].wait()

    return gk(idx_flat, table)


def _sigmoid(x):
    return 0.5 * jnp.tanh(0.5 * x) + 0.5


def _silu(x):
    u = 0.5 * x
    return u * jnp.tanh(u) + u


def _edge_node_body(e_ref, fj_ref, dk_ref, We1_ref, be1_ref, We2_ref, be2_ref,
                    WgT_ref, bg_ref, Wn1_ref, bn1_ref, Wn2_ref, bn2_ref,
                    out_ref):
    f32, bf16 = jnp.float32, jnp.bfloat16
    e = e_ref[...]
    eb = e.astype(bf16)
    We1 = We1_ref[...]
    hs = 512
    Wa_lo = We1[0:_D, 0:hs].astype(bf16)
    Wa_hi = We1[0:_D, hs:].astype(bf16)
    Wbd_lo = We1[_D:, 0:hs].astype(bf16)
    Wbd_hi = We1[_D:, hs:].astype(bf16)
    be1 = be1_ref[...]
    P_lo = jnp.dot(eb, Wa_lo, preferred_element_type=f32) + be1[:, 0:hs]
    P_hi = jnp.dot(eb, Wa_hi, preferred_element_type=f32) + be1[:, hs:]
    We2 = We2_ref[...]
    We2_lo = We2[0:hs, :].astype(bf16)
    We2_hi = We2[hs:, :].astype(bf16)
    be2 = be2_ref[...]
    WgT = WgT_ref[...]
    bg = bg_ref[...]
    msum = jnp.zeros((_RE, _M), f32)
    for k in range(_K):
        fjk = fj_ref[k]
        dkk = dk_ref[:, k:k + 1]
        fjd = jnp.concatenate([fjk.astype(bf16), dkk.astype(bf16)], axis=1)
        h_lo = _silu(P_lo + jnp.dot(fjd, Wbd_lo, preferred_element_type=f32))
        h_hi = _silu(P_hi + jnp.dot(fjd, Wbd_hi, preferred_element_type=f32))
        m = _silu(
            jnp.dot(h_lo.astype(bf16), We2_lo, preferred_element_type=f32)
            + jnp.dot(h_hi.astype(bf16), We2_hi, preferred_element_type=f32)
            + be2)
        g = _sigmoid(jnp.sum(m * WgT, axis=1, keepdims=True) + bg)
        msum = msum + m * g
    m_i = msum * f32(1.0 / _K)
    Wn1 = Wn1_ref[...]
    nh = _silu(
        jnp.dot(eb, Wn1[0:_D, :].astype(bf16), preferred_element_type=f32)
        + jnp.dot(m_i, Wn1[_D:_D + _M, :], preferred_element_type=f32)
        + bn1_ref[...]
    )
    out = jnp.dot(nh, Wn2_ref[...], preferred_element_type=f32) \
        + bn2_ref[...] + e
    out_ref[...] = out


def _edge_node_call_args():
    bn = _N
    grid = (bn // _RE,)

    def full(shape):
        return pl.BlockSpec(shape, lambda i: tuple(0 for _ in shape))

    return dict(
        grid=grid,
        in_specs=[
            pl.BlockSpec((_RE, _D), lambda i: (i, 0)),
            pl.BlockSpec((_K, _RE, _D), lambda i: (0, i, 0)),
            pl.BlockSpec((_RE, _K), lambda i: (i, 0)),
            full((2 * _D + 1, 2 * (2 * _D + 1))),
            full((1, 2 * (2 * _D + 1))),
            full((2 * (2 * _D + 1), _M)),
            full((1, _M)),
            full((1, _M)),
            full((1, 1)),
            full((_D + _M, 2 * _D)),
            full((1, 2 * _D)),
            full((2 * _D, _D)),
            full((1, _D)),
        ],
        out_specs=pl.BlockSpec((_RE, _D), lambda i: (i, 0)),
        out_shape=jax.ShapeDtypeStruct((bn, _D), jnp.float32),
    )


def kernel(emb, coors, mask, We1, be1, We2, be2, Wg, bg, Wn1, bn1, Wn2, bn2):
    coorsT = jnp.swapaxes(coors, 1, 2)
    emb2 = emb.reshape(_B * _N, _D)
    knn_args = _knn_call_args()
    edge_args = _edge_node_call_args()
    wts = (We1, be1.reshape(1, -1), We2, be2.reshape(1, -1),
           Wg.reshape(1, -1), bg.reshape(1, 1), Wn1, bn1.reshape(1, -1),
           Wn2, bn2.reshape(1, -1))
    idx_dk = [
        pl.pallas_call(_make_knn_body(b * _N), **knn_args)(
            coors[b:b + 1], coorsT[b:b + 1])
        for b in range(_B)
    ]
    outs = []
    for b in range(_B):
        idx, dk = idx_dk[b]
        idx_t = idx.reshape(_N, _K).T.reshape(-1)
        fj = _sc_gather(idx_t, emb2)
        out_b = pl.pallas_call(_edge_node_body, **edge_args)(
            emb2[b * _N:(b + 1) * _N],
            fj.reshape(_K, _N, _D),
            dk.reshape(_N, _K),
            *wts,
        )
        outs.append(out_b)
    return jnp.stack(outs), coors, mask


# speedup vs baseline: 1.0426x; 1.0426x over previous
import functools

import jax
import jax.numpy as jnp
import numpy as np
from jax import lax
from jax.experimental import pallas as pl
from jax.experimental.pallas import tpu as pltpu
from jax.experimental.pallas import tpu_sc as plsc

_B, _N, _D, _M, _K = 2, 2048, 128, 16, 16
_RK = 256
_RE = 256
_NC, _NS = 2, 16
_NW = _NC * _NS
_TOT = _B * _N * _K
_PW = _TOT // _NW
_CH = 128


def _make_knn_body(boff):
    def _knn_body(ci_ref, ct_ref, idx_ref, dk_ref):
        ci = ci_ref[0]
        ct = ct_ref[0]
        d = None
        for a in range(3):
            diff = ci[:, a:a + 1] - ct[a:a + 1, :]
            sq = diff * diff
            d = sq if d is None else d + sq
        iota = lax.broadcasted_iota(jnp.int32, (_RK, _N), 1)
        bits = lax.bitcast_convert_type(d, jnp.int32)
        p = lax.bitcast_convert_type(
            ((bits & jnp.int32(~0x7FF)) | iota) + jnp.int32(0x00800000),
            jnp.float32)
        sentinel = jnp.float32(np.inf)
        for k in range(_K):
            w = jnp.min(p, axis=1, keepdims=True)
            wb = lax.bitcast_convert_type(w, jnp.int32) \
                - jnp.int32(0x00800000)
            idx_ref[0, :, k:k + 1] = (wb & jnp.int32(0x7FF)) + boff
            dk_ref[0, :, k:k + 1] = lax.bitcast_convert_type(
                wb & jnp.int32(~0x7FF), jnp.float32)
            p = jnp.where(p == w, sentinel, p)

    return _knn_body


def _knn_call_args():
    grid = (1, _N // _RK)
    return dict(
        grid=grid,
        in_specs=[
            pl.BlockSpec((1, _RK, 3), lambda b, i: (b, i, 0)),
            pl.BlockSpec((1, 3, _N), lambda b, i: (b, 0, 0)),
        ],
        out_specs=[
            pl.BlockSpec((1, _RK, _K), lambda b, i: (b, i, 0)),
            pl.BlockSpec((1, _RK, _K), lambda b, i: (b, i, 0)),
        ],
        out_shape=[
            jax.ShapeDtypeStruct((1, _N, _K), jnp.int32),
            jax.ShapeDtypeStruct((1, _N, _K), jnp.float32),
        ],
    )


def _sc_gather(idx_flat, table):
    tot = idx_flat.shape[0]
    pw = tot // _NW
    mesh = plsc.VectorSubcoreMesh(core_axis_name="c", subcore_axis_name="s")

    @functools.partial(
        pl.kernel,
        mesh=mesh,
        out_type=jax.ShapeDtypeStruct((tot, _D), jnp.float32),
        scratch_types=[
            pltpu.VMEM((pw,), jnp.int32),
            pltpu.VMEM((_CH, _D), jnp.float32),
            pltpu.VMEM((_CH, _D), jnp.float32),
            pltpu.SemaphoreType.DMA,
            pltpu.SemaphoreType.DMA,
            pltpu.SemaphoreType.DMA,
            pltpu.SemaphoreType.DMA,
        ],
    )
    def gk(idx_hbm, tab_hbm, out_hbm, idx_v, rows0, rows1, g0, g1, w0, w1):
        wid = lax.axis_index("s") * _NC + lax.axis_index("c")
        base = wid * pw
        pltpu.sync_copy(idx_hbm.at[pl.ds(base, pw)], idx_v)
        bufs, gsems, wsems = (rows0, rows1), (g0, g1), (w0, w1)
        nch = pw // _CH
        gathers = [None] * nch
        wbacks = [None] * nch

        def start_gather(j):
            s = j & 1
            if j >= 2:
                wbacks[j - 2].wait()
            gathers[j] = pltpu.async_copy(
                tab_hbm.at[idx_v.at[pl.ds(j * _CH, _CH)]], bufs[s], gsems[s])

        start_gather(0)
        for j in range(nch):
            if j + 1 < nch:
                start_gather(j + 1)
            gathers[j].wait()
            wbacks[j] = pltpu.async_copy(
                bufs[j & 1], out_hbm.at[pl.ds(base + j * _CH, _CH)],
                wsems[j & 1])
        wbacks[nch - 2].wait()
        wbacks[nch - 1].wait()

    return gk(idx_flat, table)


def _sigmoid(x):
    return 0.5 * jnp.tanh(0.5 * x) + 0.5


def _silu(x):
    u = 0.5 * x
    return u * jnp.tanh(u) + u


def _edge_node_body(e_ref, fj_ref, dk_ref, We1_ref, be1_ref, We2_ref, be2_ref,
                    WgT_ref, bg_ref, Wn1_ref, bn1_ref, Wn2_ref, bn2_ref,
                    out_ref):
    f32, bf16 = jnp.float32, jnp.bfloat16
    e = e_ref[...]
    eb = e.astype(bf16)
    We1 = We1_ref[...]
    Wa = We1[0:_D, :].astype(bf16)
    Wbd = We1[_D:, :].astype(bf16)
    P = jnp.dot(eb, Wa, preferred_element_type=f32) + be1_ref[...]
    We2 = We2_ref[...].astype(bf16)
    be2 = be2_ref[...]
    WgT = WgT_ref[...]
    bg = bg_ref[...]
    msum = jnp.zeros((_RE, _M), f32)
    for k in range(_K):
        fjk = fj_ref[k]
        dkk = dk_ref[:, k:k + 1]
        fjd = jnp.concatenate([fjk.astype(bf16), dkk.astype(bf16)], axis=1)
        q = jnp.dot(fjd, Wbd, preferred_element_type=f32)
        h = _silu(P + q)
        m = _silu(jnp.dot(h.astype(bf16), We2, preferred_element_type=f32)
                  + be2)
        g = _sigmoid(jnp.sum(m * WgT, axis=1, keepdims=True) + bg)
        msum = msum + m * g
    m_i = msum * f32(1.0 / _K)
    Wn1 = Wn1_ref[...]
    nh = _silu(
        jnp.dot(eb, Wn1[0:_D, :].astype(bf16), preferred_element_type=f32)
        + jnp.dot(m_i, Wn1[_D:_D + _M, :], preferred_element_type=f32)
        + bn1_ref[...]
    )
    out = jnp.dot(nh, Wn2_ref[...], preferred_element_type=f32) \
        + bn2_ref[...] + e
    out_ref[...] = out


def _edge_node_call_args():
    bn = _N
    grid = (bn // _RE,)

    def full(shape):
        return pl.BlockSpec(shape, lambda i: tuple(0 for _ in shape))

    return dict(
        grid=grid,
        in_specs=[
            pl.BlockSpec((_RE, _D), lambda i: (i, 0)),
            pl.BlockSpec((_K, _RE, _D), lambda i: (0, i, 0)),
            pl.BlockSpec((_RE, _K), lambda i: (i, 0)),
            full((2 * _D + 1, 2 * (2 * _D + 1))),
            full((1, 2 * (2 * _D + 1))),
            full((2 * (2 * _D + 1), _M)),
            full((1, _M)),
            full((1, _M)),
            full((1, 1)),
            full((_D + _M, 2 * _D)),
            full((1, 2 * _D)),
            full((2 * _D, _D)),
            full((1, _D)),
        ],
        out_specs=pl.BlockSpec((_RE, _D), lambda i: (i, 0)),
        out_shape=jax.ShapeDtypeStruct((bn, _D), jnp.float32),
    )


def kernel(emb, coors, mask, We1, be1, We2, be2, Wg, bg, Wn1, bn1, Wn2, bn2):
    coorsT = jnp.swapaxes(coors, 1, 2)
    emb2 = emb.reshape(_B * _N, _D)
    knn_args = _knn_call_args()
    edge_args = _edge_node_call_args()
    wts = (We1, be1.reshape(1, -1), We2, be2.reshape(1, -1),
           Wg.reshape(1, -1), bg.reshape(1, 1), Wn1, bn1.reshape(1, -1),
           Wn2, bn2.reshape(1, -1))
    idx_dk = [
        pl.pallas_call(_make_knn_body(b * _N), **knn_args)(
            coors[b:b + 1], coorsT[b:b + 1])
        for b in range(_B)
    ]
    outs = []
    for b in range(_B):
        idx, dk = idx_dk[b]
        idx_t = idx.reshape(_N, _K).T.reshape(-1)
        fj = _sc_gather(idx_t, emb2)
        out_b = pl.pallas_call(_edge_node_body, **edge_args)(
            emb2[b * _N:(b + 1) * _N],
            fj.reshape(_K, _N, _D),
            dk.reshape(_N, _K),
            *wts,
        )
        outs.append(out_b)
    return jnp.stack(outs), coors, mask


# speedup vs baseline: 1.0447x; 1.0021x over previous
import functools

import jax
import jax.numpy as jnp
import numpy as np
from jax import lax
from jax.experimental import pallas as pl
from jax.experimental.pallas import tpu as pltpu
from jax.experimental.pallas import tpu_sc as plsc

_B, _N, _D, _M, _K = 2, 2048, 128, 16, 16
_RK = 512
_RE = 512
_NC, _NS = 2, 16
_NW = _NC * _NS
_TOT = _B * _N * _K
_PW = _TOT // _NW
_CH = 128


def _make_knn_body(boff):
    def _knn_body(ci_ref, ct_ref, idx_ref, dk_ref):
        ci = ci_ref[0]
        ct = ct_ref[0]
        d = None
        for a in range(3):
            diff = ci[:, a:a + 1] - ct[a:a + 1, :]
            sq = diff * diff
            d = sq if d is None else d + sq
        iota = lax.broadcasted_iota(jnp.int32, (_RK, _N), 1)
        bits = lax.bitcast_convert_type(d, jnp.int32)
        p = lax.bitcast_convert_type(
            ((bits & jnp.int32(~0x7FF)) | iota) + jnp.int32(0x00800000),
            jnp.float32)
        sentinel = jnp.float32(np.inf)
        for k in range(_K):
            w = jnp.min(p, axis=1, keepdims=True)
            wb = lax.bitcast_convert_type(w, jnp.int32) \
                - jnp.int32(0x00800000)
            idx_ref[0, :, k:k + 1] = (wb & jnp.int32(0x7FF)) + boff
            dk_ref[0, :, k:k + 1] = lax.bitcast_convert_type(
                wb & jnp.int32(~0x7FF), jnp.float32)
            p = jnp.where(p == w, sentinel, p)

    return _knn_body


def _knn_call_args():
    grid = (1, _N // _RK)
    return dict(
        grid=grid,
        in_specs=[
            pl.BlockSpec((1, _RK, 3), lambda b, i: (b, i, 0)),
            pl.BlockSpec((1, 3, _N), lambda b, i: (b, 0, 0)),
        ],
        out_specs=[
            pl.BlockSpec((1, _RK, _K), lambda b, i: (b, i, 0)),
            pl.BlockSpec((1, _RK, _K), lambda b, i: (b, i, 0)),
        ],
        out_shape=[
            jax.ShapeDtypeStruct((1, _N, _K), jnp.int32),
            jax.ShapeDtypeStruct((1, _N, _K), jnp.float32),
        ],
    )


def _sc_gather(idx_flat, table):
    tot = idx_flat.shape[0]
    pw = tot // _NW
    mesh = plsc.VectorSubcoreMesh(core_axis_name="c", subcore_axis_name="s")

    @functools.partial(
        pl.kernel,
        mesh=mesh,
        out_type=jax.ShapeDtypeStruct((tot, _D), jnp.float32),
        scratch_types=[
            pltpu.VMEM((pw,), jnp.int32),
            pltpu.VMEM((_CH, _D), jnp.float32),
            pltpu.VMEM((_CH, _D), jnp.float32),
            pltpu.SemaphoreType.DMA,
            pltpu.SemaphoreType.DMA,
            pltpu.SemaphoreType.DMA,
            pltpu.SemaphoreType.DMA,
        ],
    )
    def gk(idx_hbm, tab_hbm, out_hbm, idx_v, rows0, rows1, g0, g1, w0, w1):
        wid = lax.axis_index("s") * _NC + lax.axis_index("c")
        base = wid * pw
        pltpu.sync_copy(idx_hbm.at[pl.ds(base, pw)], idx_v)
        bufs, gsems, wsems = (rows0, rows1), (g0, g1), (w0, w1)
        nch = pw // _CH
        gathers = [None] * nch
        wbacks = [None] * nch

        def start_gather(j):
            s = j & 1
            if j >= 2:
                wbacks[j - 2].wait()
            gathers[j] = pltpu.async_copy(
                tab_hbm.at[idx_v.at[pl.ds(j * _CH, _CH)]], bufs[s], gsems[s])

        start_gather(0)
        for j in range(nch):
            if j + 1 < nch:
                start_gather(j + 1)
            gathers[j].wait()
            wbacks[j] = pltpu.async_copy(
                bufs[j & 1], out_hbm.at[pl.ds(base + j * _CH, _CH)],
                wsems[j & 1])
        wbacks[nch - 2].wait()
        wbacks[nch - 1].wait()

    return gk(idx_flat, table)


def _sigmoid(x):
    return 0.5 * jnp.tanh(0.5 * x) + 0.5


def _silu(x):
    u = 0.5 * x
    return u * jnp.tanh(u) + u


def _edge_node_body(e_ref, fj_ref, dk_ref, We1_ref, be1_ref, We2_ref, be2_ref,
                    WgT_ref, bg_ref, Wn1_ref, bn1_ref, Wn2_ref, bn2_ref,
                    out_ref):
    f32, bf16 = jnp.float32, jnp.bfloat16
    e = e_ref[...]
    eb = e.astype(bf16)
    We1 = We1_ref[...]
    Wa = We1[0:_D, :].astype(bf16)
    Wbd = We1[_D:, :].astype(bf16)
    P = jnp.dot(eb, Wa, preferred_element_type=f32) + be1_ref[...]
    We2 = We2_ref[...].astype(bf16)
    be2 = be2_ref[...]
    WgT = WgT_ref[...]
    bg = bg_ref[...]
    msum = jnp.zeros((_RE, _M), f32)
    for k in range(_K):
        fjk = fj_ref[k]
        dkk = dk_ref[:, k:k + 1]
        fjd = jnp.concatenate([fjk.astype(bf16), dkk.astype(bf16)], axis=1)
        q = jnp.dot(fjd, Wbd, preferred_element_type=f32)
        h = _silu(P + q)
        m = _silu(jnp.dot(h.astype(bf16), We2, preferred_element_type=f32)
                  + be2)
        g = _sigmoid(jnp.sum(m * WgT, axis=1, keepdims=True) + bg)
        msum = msum + m * g
    m_i = msum * f32(1.0 / _K)
    Wn1 = Wn1_ref[...]
    nh = _silu(
        jnp.dot(eb, Wn1[0:_D, :].astype(bf16), preferred_element_type=f32)
        + jnp.dot(m_i, Wn1[_D:_D + _M, :], preferred_element_type=f32)
        + bn1_ref[...]
    )
    out = jnp.dot(nh, Wn2_ref[...], preferred_element_type=f32) \
        + bn2_ref[...] + e
    out_ref[...] = out


def _edge_node_call_args():
    bn = _N
    grid = (bn // _RE,)

    def full(shape):
        return pl.BlockSpec(shape, lambda i: tuple(0 for _ in shape))

    return dict(
        grid=grid,
        in_specs=[
            pl.BlockSpec((_RE, _D), lambda i: (i, 0)),
            pl.BlockSpec((_K, _RE, _D), lambda i: (0, i, 0)),
            pl.BlockSpec((_RE, _K), lambda i: (i, 0)),
            full((2 * _D + 1, 2 * (2 * _D + 1))),
            full((1, 2 * (2 * _D + 1))),
            full((2 * (2 * _D + 1), _M)),
            full((1, _M)),
            full((1, _M)),
            full((1, 1)),
            full((_D + _M, 2 * _D)),
            full((1, 2 * _D)),
            full((2 * _D, _D)),
            full((1, _D)),
        ],
        out_specs=pl.BlockSpec((_RE, _D), lambda i: (i, 0)),
        out_shape=jax.ShapeDtypeStruct((bn, _D), jnp.float32),
    )


def kernel(emb, coors, mask, We1, be1, We2, be2, Wg, bg, Wn1, bn1, Wn2, bn2):
    coorsT = jnp.swapaxes(coors, 1, 2)
    emb2 = emb.reshape(_B * _N, _D)
    knn_args = _knn_call_args()
    edge_args = _edge_node_call_args()
    wts = (We1, be1.reshape(1, -1), We2, be2.reshape(1, -1),
           Wg.reshape(1, -1), bg.reshape(1, 1), Wn1, bn1.reshape(1, -1),
           Wn2, bn2.reshape(1, -1))
    idx_dk = [
        pl.pallas_call(_make_knn_body(b * _N), **knn_args)(
            coors[b:b + 1], coorsT[b:b + 1])
        for b in range(_B)
    ]
    outs = []
    for b in range(_B):
        idx, dk = idx_dk[b]
        idx_t = idx.reshape(_N, _K).T.reshape(-1)
        fj = _sc_gather(idx_t, emb2)
        out_b = pl.pallas_call(_edge_node_body, **edge_args)(
            emb2[b * _N:(b + 1) * _N],
            fj.reshape(_K, _N, _D),
            dk.reshape(_N, _K),
            *wts,
        )
        outs.append(out_b)
    return jnp.stack(outs), coors, mask


# speedup vs baseline: 1.0453x; 1.0006x over previous
import functools

import jax
import jax.numpy as jnp
import numpy as np
from jax import lax
from jax.experimental import pallas as pl
from jax.experimental.pallas import tpu as pltpu
from jax.experimental.pallas import tpu_sc as plsc

_B, _N, _D, _M, _K = 2, 2048, 128, 16, 16
_RK = 256
_RE = 256
_NC, _NS = 2, 16
_NW = _NC * _NS
_TOT = _B * _N * _K
_PW = _TOT // _NW
_CH = 128


def _make_knn_body(boff):
    def _knn_body(ci_ref, ct_ref, idx_ref, dk_ref):
        ci = ci_ref[0]
        ct = ct_ref[0]
        d = None
        for a in range(3):
            diff = ci[:, a:a + 1] - ct[a:a + 1, :]
            sq = diff * diff
            d = sq if d is None else d + sq
        iota = lax.broadcasted_iota(jnp.int32, (_RK, _N), 1)
        bits = lax.bitcast_convert_type(d, jnp.int32)
        p = lax.bitcast_convert_type(
            ((bits & jnp.int32(~0x7FF)) | iota) + jnp.int32(0x00800000),
            jnp.float32)
        sentinel = jnp.float32(np.inf)
        for k in range(_K):
            w = jnp.min(p, axis=1, keepdims=True)
            wb = lax.bitcast_convert_type(w, jnp.int32) \
                - jnp.int32(0x00800000)
            idx_ref[0, :, k:k + 1] = (wb & jnp.int32(0x7FF)) + boff
            dk_ref[0, :, k:k + 1] = lax.bitcast_convert_type(
                wb & jnp.int32(~0x7FF), jnp.float32)
            p = jnp.where(p == w, sentinel, p)

    return _knn_body


def _knn_call_args():
    grid = (1, _N // _RK)
    return dict(
        grid=grid,
        in_specs=[
            pl.BlockSpec((1, _RK, 3), lambda b, i: (b, i, 0)),
            pl.BlockSpec((1, 3, _N), lambda b, i: (b, 0, 0)),
        ],
        out_specs=[
            pl.BlockSpec((1, _RK, _K), lambda b, i: (b, i, 0)),
            pl.BlockSpec((1, _RK, _K), lambda b, i: (b, i, 0)),
        ],
        out_shape=[
            jax.ShapeDtypeStruct((1, _N, _K), jnp.int32),
            jax.ShapeDtypeStruct((1, _N, _K), jnp.float32),
        ],
    )


def _sc_gather(idx_flat, table):
    tot = idx_flat.shape[0]
    pw = tot // _NW
    mesh = plsc.VectorSubcoreMesh(core_axis_name="c", subcore_axis_name="s")

    @functools.partial(
        pl.kernel,
        mesh=mesh,
        out_type=jax.ShapeDtypeStruct((tot, _D), jnp.float32),
        scratch_types=[
            pltpu.VMEM((pw,), jnp.int32),
            pltpu.VMEM((_CH, _D), jnp.float32),
            pltpu.VMEM((_CH, _D), jnp.float32),
            pltpu.SemaphoreType.DMA,
            pltpu.SemaphoreType.DMA,
            pltpu.SemaphoreType.DMA,
            pltpu.SemaphoreType.DMA,
        ],
    )
    def gk(idx_hbm, tab_hbm, out_hbm, idx_v, rows0, rows1, g0, g1, w0, w1):
        wid = lax.axis_index("s") * _NC + lax.axis_index("c")
        base = wid * pw
        pltpu.sync_copy(idx_hbm.at[pl.ds(base, pw)], idx_v)
        bufs, gsems, wsems = (rows0, rows1), (g0, g1), (w0, w1)
        nch = pw // _CH
        gathers = [None] * nch
        wbacks = [None] * nch

        def start_gather(j):
            s = j & 1
            if j >= 2:
                wbacks[j - 2].wait()
            gathers[j] = pltpu.async_copy(
                tab_hbm.at[idx_v.at[pl.ds(j * _CH, _CH)]], bufs[s], gsems[s])

        start_gather(0)
        for j in range(nch):
            if j + 1 < nch:
                start_gather(j + 1)
            gathers[j].wait()
            wbacks[j] = pltpu.async_copy(
                bufs[j & 1], out_hbm.at[pl.ds(base + j * _CH, _CH)],
                wsems[j & 1])
        wbacks[nch - 2].wait()
        wbacks[nch - 1].wait()

    return gk(idx_flat, table)


def _sigmoid(x):
    return 0.5 * jnp.tanh(0.5 * x) + 0.5


def _silu(x):
    u = 0.5 * x
    return u * jnp.tanh(u) + u


def _edge_node_body(e_ref, fj_ref, dk_ref, We1_ref, be1_ref, We2_ref, be2_ref,
                    WgT_ref, bg_ref, Wn1_ref, bn1_ref, Wn2_ref, bn2_ref,
                    out_ref):
    f32, bf16 = jnp.float32, jnp.bfloat16
    e = e_ref[...]
    eb = e.astype(bf16)
    We1 = We1_ref[...]
    Wa = We1[0:_D, :].astype(bf16)
    Wbd = We1[_D:, :].astype(bf16)
    P = jnp.dot(eb, Wa, preferred_element_type=f32) + be1_ref[...]
    We2 = We2_ref[...].astype(bf16)
    be2 = be2_ref[...]
    WgT = WgT_ref[...]
    bg = bg_ref[...]
    msum = jnp.zeros((_RE, _M), f32)
    for k in range(_K):
        fjk = fj_ref[k]
        dkk = dk_ref[:, k:k + 1]
        fjd = jnp.concatenate([fjk.astype(bf16), dkk.astype(bf16)], axis=1)
        q = jnp.dot(fjd, Wbd, preferred_element_type=f32)
        h = _silu(P + q)
        m = _silu(jnp.dot(h.astype(bf16), We2, preferred_element_type=f32)
                  + be2)
        g = _sigmoid(jnp.sum(m * WgT, axis=1, keepdims=True) + bg)
        msum = msum + m * g
    m_i = msum * f32(1.0 / _K)
    Wn1 = Wn1_ref[...]
    nh = _silu(
        jnp.dot(eb, Wn1[0:_D, :].astype(bf16), preferred_element_type=f32)
        + jnp.dot(m_i, Wn1[_D:_D + _M, :], preferred_element_type=f32)
        + bn1_ref[...]
    )
    out = jnp.dot(nh, Wn2_ref[...], preferred_element_type=f32) \
        + bn2_ref[...] + e
    out_ref[...] = out


def _edge_node_call_args():
    bn = _N
    grid = (bn // _RE,)

    def full(shape):
        return pl.BlockSpec(shape, lambda i: tuple(0 for _ in shape))

    return dict(
        grid=grid,
        in_specs=[
            pl.BlockSpec((_RE, _D), lambda i: (i, 0)),
            pl.BlockSpec((_K, _RE, _D), lambda i: (0, i, 0)),
            pl.BlockSpec((_RE, _K), lambda i: (i, 0)),
            full((2 * _D + 1, 2 * (2 * _D + 1))),
            full((1, 2 * (2 * _D + 1))),
            full((2 * (2 * _D + 1), _M)),
            full((1, _M)),
            full((1, _M)),
            full((1, 1)),
            full((_D + _M, 2 * _D)),
            full((1, 2 * _D)),
            full((2 * _D, _D)),
            full((1, _D)),
        ],
        out_specs=pl.BlockSpec((_RE, _D), lambda i: (i, 0)),
        out_shape=jax.ShapeDtypeStruct((bn, _D), jnp.float32),
    )


def kernel(emb, coors, mask, We1, be1, We2, be2, Wg, bg, Wn1, bn1, Wn2, bn2):
    coorsT = jnp.swapaxes(coors, 1, 2)
    emb2 = emb.reshape(_B * _N, _D)
    knn_args = _knn_call_args()
    edge_args = _edge_node_call_args()
    wts = (We1, be1.reshape(1, -1), We2, be2.reshape(1, -1),
           Wg.reshape(1, -1), bg.reshape(1, 1), Wn1, bn1.reshape(1, -1),
           Wn2, bn2.reshape(1, -1))
    idx_dk = [
        pl.pallas_call(_make_knn_body(b * _N), **knn_args)(
            coors[b:b + 1], coorsT[b:b + 1])
        for b in range(_B)
    ]
    outs = []
    for b in range(_B):
        idx, dk = idx_dk[b]
        idx_t = idx.reshape(_N, _K).T.reshape(-1)
        fj = _sc_gather(idx_t, emb2)
        out_b = pl.pallas_call(_edge_node_body, **edge_args)(
            emb2[b * _N:(b + 1) * _N],
            fj.reshape(_K, _N, _D),
            dk.reshape(_N, _K),
            *wts,
        )
        outs.append(out_b)
    return jnp.stack(outs), coors, mask


# speedup vs baseline: 1.0759x; 1.0292x over previous
import functools

import jax
import jax.numpy as jnp
import numpy as np
from jax import lax
from jax.experimental import pallas as pl
from jax.experimental.pallas import tpu as pltpu
from jax.experimental.pallas import tpu_sc as plsc

_B, _N, _D, _M, _K = 2, 2048, 128, 16, 16
_RK = 256
_RE = 256
_NC, _NS = 2, 16
_NW = _NC * _NS
_TOT = _B * _N * _K
_PW = _TOT // _NW
_CH = 128


def _make_knn_body(boff):
    def _knn_body(ci_ref, ct_ref, idx_ref, dk_ref):
        ci = ci_ref[0]
        ct = ct_ref[0]
        d = None
        for a in range(3):
            diff = ci[:, a:a + 1] - ct[a:a + 1, :]
            sq = diff * diff
            d = sq if d is None else d + sq
        iota = lax.broadcasted_iota(jnp.int32, (_RK, _N), 1)
        bits = lax.bitcast_convert_type(d, jnp.int32)
        p = lax.bitcast_convert_type(
            ((bits & jnp.int32(~0x7FF)) | iota) + jnp.int32(0x00800000),
            jnp.float32)
        sentinel = jnp.float32(np.inf)
        for k in range(_K):
            w = jnp.min(p, axis=1, keepdims=True)
            wb = lax.bitcast_convert_type(w, jnp.int32) \
                - jnp.int32(0x00800000)
            idx_ref[0, :, k:k + 1] = (wb & jnp.int32(0x7FF)) + boff
            dk_ref[0, :, k:k + 1] = lax.bitcast_convert_type(
                wb & jnp.int32(~0x7FF), jnp.float32)
            p = jnp.where(p == w, sentinel, p)

    return _knn_body


def _knn_call_args():
    grid = (1, _N // _RK)
    return dict(
        grid=grid,
        in_specs=[
            pl.BlockSpec((1, _RK, 3), lambda b, i: (b, i, 0)),
            pl.BlockSpec((1, 3, _N), lambda b, i: (b, 0, 0)),
        ],
        out_specs=[
            pl.BlockSpec((1, _RK, _K), lambda b, i: (b, i, 0)),
            pl.BlockSpec((1, _RK, _K), lambda b, i: (b, i, 0)),
        ],
        out_shape=[
            jax.ShapeDtypeStruct((1, _N, _K), jnp.int32),
            jax.ShapeDtypeStruct((1, _N, _K), jnp.float32),
        ],
    )


def _sc_gather(idx_flat, table):
    tot = idx_flat.shape[0]
    pw = tot // _NW
    mesh = plsc.VectorSubcoreMesh(core_axis_name="c", subcore_axis_name="s")

    @functools.partial(
        pl.kernel,
        mesh=mesh,
        out_type=jax.ShapeDtypeStruct((tot, _D), jnp.float32),
        scratch_types=[
            pltpu.VMEM((pw,), jnp.int32),
            pltpu.VMEM((_CH, _D), jnp.float32),
            pltpu.VMEM((_CH, _D), jnp.float32),
            pltpu.SemaphoreType.DMA,
            pltpu.SemaphoreType.DMA,
            pltpu.SemaphoreType.DMA,
            pltpu.SemaphoreType.DMA,
        ],
    )
    def gk(idx_hbm, tab_hbm, out_hbm, idx_v, rows0, rows1, g0, g1, w0, w1):
        wid = lax.axis_index("s") * _NC + lax.axis_index("c")
        base = wid * pw
        pltpu.sync_copy(idx_hbm.at[pl.ds(base, pw)], idx_v)
        bufs, gsems, wsems = (rows0, rows1), (g0, g1), (w0, w1)
        nch = pw // _CH
        gathers = [None] * nch
        wbacks = [None] * nch

        def start_gather(j):
            s = j & 1
            if j >= 2:
                wbacks[j - 2].wait()
            gathers[j] = pltpu.async_copy(
                tab_hbm.at[idx_v.at[pl.ds(j * _CH, _CH)]], bufs[s], gsems[s])

        start_gather(0)
        for j in range(nch):
            if j + 1 < nch:
                start_gather(j + 1)
            gathers[j].wait()
            wbacks[j] = pltpu.async_copy(
                bufs[j & 1], out_hbm.at[pl.ds(base + j * _CH, _CH)],
                wsems[j & 1])
        wbacks[nch - 2].wait()
        wbacks[nch - 1].wait()

    return gk(idx_flat, table)


def _sigmoid(x):
    return 0.5 * jnp.tanh(0.5 * x) + 0.5


def _silu(x):
    u = 0.5 * x
    return u * jnp.tanh(u) + u


def _edge_node_body(e_ref, fj_ref, dk_ref, We1_ref, be1_ref, We2_ref, be2_ref,
                    WgT_ref, bg_ref, Wn1_ref, bn1_ref, Wn2_ref, bn2_ref,
                    out_ref):
    f32, bf16 = jnp.float32, jnp.bfloat16
    e = e_ref[...]
    eb = e.astype(bf16)
    We1 = We1_ref[...]
    Wa = We1[0:_D, :].astype(bf16)
    Wbd = We1[_D:, :].astype(bf16)
    P = jnp.dot(eb, Wa, preferred_element_type=f32) + be1_ref[...]
    We2 = We2_ref[...].astype(bf16)
    be2 = be2_ref[...]
    WgT = WgT_ref[...]
    bg = bg_ref[...]
    msum = jnp.zeros((_RE, _M), f32)
    for k in range(_K):
        fjk = fj_ref[k]
        dkk = dk_ref[:, k:k + 1]
        fjd = jnp.concatenate([fjk.astype(bf16), dkk.astype(bf16)], axis=1)
        q = jnp.dot(fjd, Wbd, preferred_element_type=f32)
        h = _silu(P + q)
        m = _silu(jnp.dot(h.astype(bf16), We2, preferred_element_type=f32)
                  + be2)
        g = _sigmoid(jnp.sum(m * WgT, axis=1, keepdims=True) + bg)
        msum = msum + m * g
    m_i = msum * f32(1.0 / _K)
    Wn1 = Wn1_ref[...]
    nh = _silu(
        jnp.dot(eb, Wn1[0:_D, :].astype(bf16), preferred_element_type=f32)
        + jnp.dot(m_i, Wn1[_D:_D + _M, :], preferred_element_type=f32)
        + bn1_ref[...]
    )
    out = jnp.dot(nh, Wn2_ref[...], preferred_element_type=f32) \
        + bn2_ref[...] + e
    out_ref[...] = out


def _edge_node_call_args(b):
    bn = _N
    grid = (bn // _RE,)
    bo = b * (_N // _RE)

    def full(shape):
        return pl.BlockSpec(shape, lambda i: tuple(0 for _ in shape))

    return dict(
        grid=grid,
        in_specs=[
            pl.BlockSpec((_RE, _D), lambda i: (i + bo, 0)),
            pl.BlockSpec((_K, _RE, _D), lambda i: (0, i, 0)),
            pl.BlockSpec((_RE, _K), lambda i: (i, 0)),
            full((2 * _D + 1, 2 * (2 * _D + 1))),
            full((1, 2 * (2 * _D + 1))),
            full((2 * (2 * _D + 1), _M)),
            full((1, _M)),
            full((1, _M)),
            full((1, 1)),
            full((_D + _M, 2 * _D)),
            full((1, 2 * _D)),
            full((2 * _D, _D)),
            full((1, _D)),
        ],
        out_specs=pl.BlockSpec((_RE, _D), lambda i: (i, 0)),
        out_shape=jax.ShapeDtypeStruct((bn, _D), jnp.float32),
    )


def kernel(emb, coors, mask, We1, be1, We2, be2, Wg, bg, Wn1, bn1, Wn2, bn2):
    coorsT = jnp.swapaxes(coors, 1, 2)
    emb2 = emb.reshape(_B * _N, _D)
    knn_args = _knn_call_args()
    wts = (We1, be1.reshape(1, -1), We2, be2.reshape(1, -1),
           Wg.reshape(1, -1), bg.reshape(1, 1), Wn1, bn1.reshape(1, -1),
           Wn2, bn2.reshape(1, -1))
    idx_dk = [
        pl.pallas_call(_make_knn_body(b * _N), **knn_args)(
            coors[b:b + 1], coorsT[b:b + 1])
        for b in range(_B)
    ]
    outs = []
    for b in range(_B):
        idx, dk = idx_dk[b]
        idx_t = idx.reshape(_N, _K).T.reshape(-1)
        fj = _sc_gather(idx_t, emb2)
        out_b = pl.pallas_call(_edge_node_body, **_edge_node_call_args(b))(
            emb2,
            fj.reshape(_K, _N, _D),
            dk.reshape(_N, _K),
            *wts,
        )
        outs.append(out_b)
    return jnp.stack(outs), coors, mask


# speedup vs baseline: 1.0794x; 1.0033x over previous
import functools

import jax
import jax.numpy as jnp
import numpy as np
from jax import lax
from jax.experimental import pallas as pl
from jax.experimental.pallas import tpu as pltpu
from jax.experimental.pallas import tpu_sc as plsc

_B, _N, _D, _M, _K = 2, 2048, 128, 16, 16
_RK = 256
_RE = 256
_NC, _NS = 2, 16
_NW = _NC * _NS
_TOT = _B * _N * _K
_PW = _TOT // _NW
_CH = 128


def _make_knn_body(boff):
    def _knn_body(ci_ref, ct_ref, idx_ref, dk_ref):
        ci = ci_ref[0]
        ct = ct_ref[0]
        d = None
        for a in range(3):
            diff = ci[:, a:a + 1] - ct[a:a + 1, :]
            sq = diff * diff
            d = sq if d is None else d + sq
        iota = lax.broadcasted_iota(jnp.int32, (_RK, _N), 1)
        bits = lax.bitcast_convert_type(d, jnp.int32)
        p = lax.bitcast_convert_type(
            ((bits & jnp.int32(~0x7FF)) | iota) + jnp.int32(0x00800000),
            jnp.float32)
        sentinel = jnp.float32(np.inf)
        for k in range(_K):
            w = jnp.min(p, axis=1, keepdims=True)
            wb = lax.bitcast_convert_type(w, jnp.int32) \
                - jnp.int32(0x00800000)
            idx_ref[0, :, k:k + 1] = (wb & jnp.int32(0x7FF)) + boff
            dk_ref[0, :, k:k + 1] = lax.bitcast_convert_type(
                wb & jnp.int32(~0x7FF), jnp.float32)
            p = jnp.where(p == w, sentinel, p)

    return _knn_body


def _knn_call_args():
    grid = (1, _N // _RK)
    return dict(
        grid=grid,
        in_specs=[
            pl.BlockSpec((1, _RK, 3), lambda b, i: (b, i, 0)),
            pl.BlockSpec((1, 3, _N), lambda b, i: (b, 0, 0)),
        ],
        out_specs=[
            pl.BlockSpec((1, _RK, _K), lambda b, i: (b, i, 0)),
            pl.BlockSpec((1, _RK, _K), lambda b, i: (b, i, 0)),
        ],
        out_shape=[
            jax.ShapeDtypeStruct((1, _N, _K), jnp.int32),
            jax.ShapeDtypeStruct((1, _N, _K), jnp.float32),
        ],
    )


def _sc_gather(idx_flat, table):
    tot = idx_flat.shape[0]
    pw = tot // _NW
    mesh = plsc.VectorSubcoreMesh(core_axis_name="c", subcore_axis_name="s")

    @functools.partial(
        pl.kernel,
        mesh=mesh,
        out_type=jax.ShapeDtypeStruct((tot, _D), jnp.float32),
        scratch_types=[
            pltpu.VMEM((pw,), jnp.int32),
            pltpu.VMEM((_CH, _D), jnp.float32),
            pltpu.VMEM((_CH, _D), jnp.float32),
            pltpu.SemaphoreType.DMA,
            pltpu.SemaphoreType.DMA,
            pltpu.SemaphoreType.DMA,
            pltpu.SemaphoreType.DMA,
        ],
    )
    def gk(idx_hbm, tab_hbm, out_hbm, idx_v, rows0, rows1, g0, g1, w0, w1):
        wid = lax.axis_index("s") * _NC + lax.axis_index("c")
        base = wid * pw
        pltpu.sync_copy(idx_hbm.at[pl.ds(base, pw)], idx_v)
        bufs, gsems, wsems = (rows0, rows1), (g0, g1), (w0, w1)
        nch = pw // _CH
        gathers = [None] * nch
        wbacks = [None] * nch

        def start_gather(j):
            s = j & 1
            if j >= 2:
                wbacks[j - 2].wait()
            gathers[j] = pltpu.async_copy(
                tab_hbm.at[idx_v.at[pl.ds(j * _CH, _CH)]], bufs[s], gsems[s])

        start_gather(0)
        for j in range(nch):
            if j + 1 < nch:
                start_gather(j + 1)
            gathers[j].wait()
            wbacks[j] = pltpu.async_copy(
                bufs[j & 1], out_hbm.at[pl.ds(base + j * _CH, _CH)],
                wsems[j & 1])
        wbacks[nch - 2].wait()
        wbacks[nch - 1].wait()

    return gk(idx_flat, table)


def _sigmoid(x):
    return 0.5 * jnp.tanh(0.5 * x) + 0.5


def _silu(x):
    u = 0.5 * x
    return u * jnp.tanh(u) + u


def _edge_node_body(e_ref, fj_ref, dk_ref, We1_ref, be1_ref, We2_ref, be2_ref,
                    WgT_ref, bg_ref, Wn1_ref, bn1_ref, Wn2_ref, bn2_ref,
                    out_ref):
    f32, bf16 = jnp.float32, jnp.bfloat16
    e = e_ref[...]
    eb = e.astype(bf16)
    We1 = We1_ref[...]
    Wa = We1[0:_D, :].astype(bf16)
    Wbd = We1[_D:, :].astype(bf16)
    P = jnp.dot(eb, Wa, preferred_element_type=f32) + be1_ref[...]
    We2 = We2_ref[...].astype(bf16)
    be2 = be2_ref[...]
    WgT = WgT_ref[...]
    bg = bg_ref[...]
    msum = jnp.zeros((_RE, _M), f32)
    for k in range(_K):
        fjk = fj_ref[k]
        dkk = dk_ref[:, k:k + 1]
        fjd = jnp.concatenate([fjk.astype(bf16), dkk.astype(bf16)], axis=1)
        q = jnp.dot(fjd, Wbd, preferred_element_type=f32)
        h = _silu(P + q)
        m = _silu(jnp.dot(h.astype(bf16), We2, preferred_element_type=f32)
                  + be2)
        g = _sigmoid(jnp.sum(m * WgT, axis=1, keepdims=True) + bg)
        msum = msum + m * g
    m_i = msum * f32(1.0 / _K)
    Wn1 = Wn1_ref[...]
    nh = _silu(
        jnp.dot(eb, Wn1[0:_D, :].astype(bf16), preferred_element_type=f32)
        + jnp.dot(m_i, Wn1[_D:_D + _M, :], preferred_element_type=f32)
        + bn1_ref[...]
    )
    out = jnp.dot(nh, Wn2_ref[...], preferred_element_type=f32) \
        + bn2_ref[...] + e
    out_ref[...] = out


def _edge_node_call_args(b):
    bn = _N
    grid = (bn // _RE,)
    bo = b * (_N // _RE)

    def full(shape):
        return pl.BlockSpec(shape, lambda i: tuple(0 for _ in shape))

    return dict(
        grid=grid,
        in_specs=[
            pl.BlockSpec((_RE, _D), lambda i: (i + bo, 0)),
            pl.BlockSpec((_K, _RE, _D), lambda i: (0, i, 0)),
            pl.BlockSpec((_RE, _K), lambda i: (i, 0)),
            full((2 * _D + 1, 2 * (2 * _D + 1))),
            full((1, 2 * (2 * _D + 1))),
            full((2 * (2 * _D + 1), _M)),
            full((1, _M)),
            full((1, _M)),
            full((1, 1)),
            full((_D + _M, 2 * _D)),
            full((1, 2 * _D)),
            full((2 * _D, _D)),
            full((1, _D)),
        ],
        out_specs=pl.BlockSpec((_RE, _D), lambda i: (i, 0)),
        out_shape=jax.ShapeDtypeStruct((bn, _D), jnp.float32),
    )


def kernel(emb, coors, mask, We1, be1, We2, be2, Wg, bg, Wn1, bn1, Wn2, bn2):
    coorsT = jnp.swapaxes(coors, 1, 2)
    emb2 = emb.reshape(_B * _N, _D)
    knn_args = _knn_call_args()
    wts = (We1, be1.reshape(1, -1), We2, be2.reshape(1, -1),
           Wg.reshape(1, -1), bg.reshape(1, 1), Wn1, bn1.reshape(1, -1),
           Wn2, bn2.reshape(1, -1))
    idx_dk = [
        pl.pallas_call(_make_knn_body(b * _N), **knn_args)(
            coors[b:b + 1], coorsT[b:b + 1])
        for b in range(_B)
    ]
    outs = []
    for b in range(_B):
        idx, dk = idx_dk[b]
        idx_t = idx.reshape(_N, _K).T.reshape(-1)
        fj = _sc_gather(idx_t, emb2)
        out_b = pl.pallas_call(_edge_node_body, **_edge_node_call_args(b))(
            emb2,
            fj.reshape(_K, _N, _D),
            dk.reshape(_N, _K),
            *wts,
        )
        outs.append(out_b)
    out = jnp.concatenate(outs, axis=0).reshape(_B, _N, _D)
    return out, coors, mask
